# baseline (speedup 1.0000x reference)
import functools
import jax
import jax.numpy as jnp
import numpy as np
from jax import lax
from jax.experimental import pallas as pl
from jax.experimental.pallas import tpu as pltpu
from jax.experimental.pallas import tpu_sc as plsc

N_NODES = 10000
N_EDGES = 320000
D = 128
H = 8
DH = 16
OUT = 64

ROW_BLK = 1000
N_ROW_BLKS = N_NODES // ROW_BLK


def _pre_body(h_ref, W_ref, alS_ref, arS_ref, featx_ref, erx_ref):
    h = h_ref[...]
    W = W_ref[0]
    feat = jnp.dot(h, W, preferred_element_type=jnp.float32)
    elx = jnp.dot(feat, alS_ref[0], preferred_element_type=jnp.float32)
    featx_ref[0] = jnp.concatenate([feat, elx], axis=1)
    erx_ref[0] = jnp.dot(feat, arS_ref[0], preferred_element_type=jnp.float32)


def _tc_pre(h, Ws, alS, arS):
    return pl.pallas_call(
        _pre_body,
        grid=(2, N_ROW_BLKS),
        in_specs=[
            pl.BlockSpec((ROW_BLK, D), lambda p, i: (i, 0)),
            pl.BlockSpec((1, D, D), lambda p, i: (p, 0, 0)),
            pl.BlockSpec((1, D, 2 * H), lambda p, i: (p, 0, 0)),
            pl.BlockSpec((1, D, 2 * H), lambda p, i: (p, 0, 0)),
        ],
        out_specs=[
            pl.BlockSpec((1, ROW_BLK, D + 2 * H), lambda p, i: (p, i, 0)),
            pl.BlockSpec((1, ROW_BLK, 2 * H), lambda p, i: (p, i, 0)),
        ],
        out_shape=[
            jax.ShapeDtypeStruct((2, N_NODES, D + 2 * H), jnp.float32),
            jax.ShapeDtypeStruct((2, N_NODES, 2 * H), jnp.float32),
        ],
    )(h, Ws, alS, arS)


def _post_body(acc_ref, Rsel_ref, sW1_ref, sb1_ref, w2_ref,
               pW_ref, pb_ref, out_ref):
    Rsel = Rsel_ref[...]
    sW1 = sW1_ref[...]
    sb1 = sb1_ref[...]
    w2 = w2_ref[...]

    def make_z(p):
        blk = acc_ref[p]
        s_rep = jnp.dot(blk[:, D:], Rsel, preferred_element_type=jnp.float32)
        x = blk[:, :D] / (s_rep + 1e-9)
        return jnp.where(x > 0, x, jnp.exp(x) - 1.0)

    z0 = make_z(0)
    z1 = make_z(1)

    def sem_w(z):
        t = jnp.tanh(jnp.dot(z, sW1, preferred_element_type=jnp.float32) + sb1)
        return (t * w2).sum(axis=1, keepdims=True)

    w0 = sem_w(z0)
    w1 = sem_w(z1)
    m = jnp.maximum(w0, w1)
    e0 = jnp.exp(w0 - m)
    e1 = jnp.exp(w1 - m)
    b0 = e0 / (e0 + e1)
    hz = b0 * z0 + (1.0 - b0) * z1
    logits = jnp.dot(hz, pW_ref[...], preferred_element_type=jnp.float32) + pb_ref[...]
    lm = jnp.max(logits, axis=1, keepdims=True)
    lse = jnp.log(jnp.sum(jnp.exp(logits - lm), axis=1, keepdims=True)) + lm
    out_ref[...] = logits - lse


def _tc_post(acc, Rsel, sem_W1, sem_b1, w2row, pred_W, pred_b):
    return pl.pallas_call(
        _post_body,
        grid=(N_ROW_BLKS,),
        in_specs=[
            pl.BlockSpec((2, ROW_BLK, D + 2 * H), lambda i: (0, i, 0)),
            pl.BlockSpec((2 * H, D), lambda i: (0, 0)),
            pl.BlockSpec((D, D), lambda i: (0, 0)),
            pl.BlockSpec((1, D), lambda i: (0, 0)),
            pl.BlockSpec((1, D), lambda i: (0, 0)),
            pl.BlockSpec((D, OUT), lambda i: (0, 0)),
            pl.BlockSpec((1, OUT), lambda i: (0, 0)),
        ],
        out_specs=pl.BlockSpec((ROW_BLK, OUT), lambda i: (i, 0)),
        out_shape=jax.ShapeDtypeStruct((N_NODES, OUT), jnp.float32),
    )(acc, Rsel, sem_W1, sem_b1, w2row, pred_W, pred_b)


CH = 80
CHUNKS_PER_CORE = N_EDGES // CH
TILES = 16
R_MAIN = 624
ZR = 312


def _edge_sc_body(featx2, erx2, idx3, zerX,
                  accx_out,
                  idx_v0, idx_v1, idx_v2, er_v0, er_v1, er_v2,
                  fx_v0, fx_v1, fx_v2,
                  accS, sem0, sem1, sem2, sco0, sco1, sco2):
    idx_v = (idx_v0, idx_v1, idx_v2)
    er_v = (er_v0, er_v1, er_v2)
    fx_v = (fx_v0, fx_v1, fx_v2)
    sems = (sem0, sem1, sem2)
    scat_sems = (sco0, sco1, sco2)
    c = lax.axis_index("c")
    s = lax.axis_index("s")
    row0 = s * R_MAIN

    for k in range(2):
        pltpu.sync_copy(zerX, accS.at[pl.ds(row0 + k * ZR, ZR), :])

    @pl.when(s == TILES - 1)
    def _zero_tail():
        pltpu.sync_copy(zerX.at[pl.ds(0, 16), :],
                        accS.at[pl.ds(TILES * R_MAIN, 16), :])

    plsc.subcore_barrier()

    nj = CHUNKS_PER_CORE // TILES

    def gather_copies(b):
        return (pltpu.make_async_copy(erx2.at[idx_v[b].at[1]], er_v[b], sems[b]),
                pltpu.make_async_copy(featx2.at[idx_v[b].at[0]], fx_v[b], sems[b]))

    def scatter_start(b):
        pltpu.async_copy(fx_v[b], accS.at[idx_v[b].at[2]], scat_sems[b], add=True)

    def scatter_wait(b):
        pltpu.make_async_copy(fx_v[b], accS.at[idx_v[b].at[2]],
                              scat_sems[b]).wait()

    def fire(step, b):
        @pl.when(step >= 3)
        def _w():
            scatter_wait(b)

        k = c * CHUNKS_PER_CORE + s + step * TILES
        pltpu.sync_copy(idx3.at[k], idx_v[b])
        for cp in gather_copies(b):
            cp.start()

    def consume(step, b):
        for cp in gather_copies(b):
            cp.wait()

        @plsc.parallel_loop(0, CH, unroll=4)
        def edge_body(i):
            x = fx_v[b][i, pl.ds(D, 2 * H)] + er_v[b][i]
            x = jnp.where(x >= 0, x, 0.2 * x)
            ex = jnp.exp(x)
            fx_v[b][i, pl.ds(D, 2 * H)] = ex
            for hh in range(H):
                idx = jnp.full((16, 1), hh, dtype=jnp.int32)
                g = lax.gather(
                    ex, idx,
                    lax.GatherDimensionNumbers(offset_dims=(),
                                               collapsed_slice_dims=(0,),
                                               start_index_map=(0,)),
                    slice_sizes=(1,),
                    mode=lax.GatherScatterMode.PROMISE_IN_BOUNDS)
                fx_v[b][i, pl.ds(hh * DH, DH)] = fx_v[b][i, pl.ds(hh * DH, DH)] * g

        scatter_start(b)

    fire(0, 0)
    fire(1, 1)

    def tri_body(jj, carry):
        for b in range(3):
            step = 3 * jj + b

            @pl.when(step + 2 < nj)
            def _f():
                fire(step + 2, (b + 2) % 3)

            @pl.when(step < nj)
            def _c():
                consume(step, b)
        return carry

    lax.fori_loop(0, (nj + 2) // 3, tri_body, 0)

    scatter_wait(0)
    scatter_wait(1)
    scatter_wait(2)

    plsc.subcore_barrier()
    out_row0 = c * N_NODES + row0
    pltpu.sync_copy(accS.at[pl.ds(row0, R_MAIN), :],
                    accx_out.at[pl.ds(out_row0, R_MAIN), :])

    @pl.when(s == TILES - 1)
    def _write_tail():
        t0 = TILES * R_MAIN
        pltpu.sync_copy(accS.at[pl.ds(t0, 16), :],
                        accx_out.at[pl.ds(c * N_NODES + t0, 16), :])


def _edge_pass_sc(featx2, erx2, idx3):
    DX = D + 2 * H
    zerX = jnp.zeros((ZR, DX), jnp.float32)
    mesh = plsc.VectorSubcoreMesh(core_axis_name="c", subcore_axis_name="s")
    f = functools.partial(
        pl.kernel,
        mesh=mesh,
        compiler_params=pltpu.CompilerParams(use_tc_tiling_on_sc=False),
        out_type=[
            jax.ShapeDtypeStruct((2 * N_NODES, DX), jnp.float32),
        ],
        scratch_types=[
            pltpu.VMEM((3, CH), jnp.int32),
            pltpu.VMEM((3, CH), jnp.int32),
            pltpu.VMEM((3, CH), jnp.int32),
            pltpu.VMEM((CH, 2 * H), jnp.float32),
            pltpu.VMEM((CH, 2 * H), jnp.float32),
            pltpu.VMEM((CH, 2 * H), jnp.float32),
            pltpu.VMEM((CH, DX), jnp.float32),
            pltpu.VMEM((CH, DX), jnp.float32),
            pltpu.VMEM((CH, DX), jnp.float32),
            pltpu.VMEM_SHARED((N_NODES, DX), jnp.float32),
            pltpu.SemaphoreType.DMA,
            pltpu.SemaphoreType.DMA,
            pltpu.SemaphoreType.DMA,
            pltpu.SemaphoreType.DMA,
            pltpu.SemaphoreType.DMA,
            pltpu.SemaphoreType.DMA,
        ],
    )(_edge_sc_body)
    return f(featx2, erx2, idx3, zerX)[0]


def kernel(h, edge_index_0, edge_index_1, fc_W_0, attn_l_0, attn_r_0,
           fc_W_1, attn_l_1, attn_r_1, sem_W1, sem_b1, sem_W2, sem_b2,
           pred_W, pred_b):
    Ws = jnp.stack([fc_W_0, fc_W_1])
    d_ids = np.arange(D) // DH
    sel = (d_ids[:, None] == (np.arange(2 * H)[None, :] % H)).astype(np.float32)
    sel = jnp.asarray(sel)
    alS = jnp.stack([attn_l_0.reshape(D, 1) * sel, attn_l_1.reshape(D, 1) * sel])
    arS = jnp.stack([attn_r_0.reshape(D, 1) * sel, attn_r_1.reshape(D, 1) * sel])

    srcg = jnp.concatenate([edge_index_0[0], edge_index_1[0] + N_NODES]).astype(jnp.int32)
    dstg = jnp.concatenate([edge_index_0[1], edge_index_1[1] + N_NODES]).astype(jnp.int32)
    dstl = jnp.concatenate([edge_index_0[1], edge_index_1[1]]).astype(jnp.int32)
    idx3 = jnp.stack([srcg.reshape(-1, CH), dstg.reshape(-1, CH),
                      dstl.reshape(-1, CH)], axis=1)

    featx, erx = _tc_pre(h, Ws, alS, arS)

    acc2 = _edge_pass_sc(featx.reshape(2 * N_NODES, D + 2 * H),
                         erx.reshape(2 * N_NODES, 2 * H), idx3)
    acc = acc2.reshape(2, N_NODES, D + 2 * H)

    Rsel = (np.arange(2 * H)[:, None] == (np.arange(D)[None, :] // DH)).astype(np.float32)
    Rsel = jnp.asarray(Rsel)
    return _tc_post(acc, Rsel, sem_W1, sem_b1.reshape(1, D),
                    sem_W2.reshape(1, D), pred_W, pred_b.reshape(1, OUT))


# speedup vs baseline: 1.0182x; 1.0182x over previous
import functools
import jax
import jax.numpy as jnp
import numpy as np
from jax import lax
from jax.experimental import pallas as pl
from jax.experimental.pallas import tpu as pltpu
from jax.experimental.pallas import tpu_sc as plsc

N_NODES = 10000
N_EDGES = 320000
D = 128
H = 8
DH = 16
OUT = 64

ROW_BLK = 1000
N_ROW_BLKS = N_NODES // ROW_BLK


def _pre_body(h_ref, W_ref, alS_ref, arS_ref, featx_ref, erx_ref):
    h = h_ref[...]
    W = W_ref[0]
    feat = jnp.dot(h, W, preferred_element_type=jnp.float32)
    elx = jnp.dot(feat, alS_ref[0], preferred_element_type=jnp.float32)
    featx_ref[0] = jnp.concatenate([feat, elx], axis=1)
    erx_ref[0] = jnp.dot(feat, arS_ref[0], preferred_element_type=jnp.float32)


def _tc_pre(h, Ws, alS, arS):
    return pl.pallas_call(
        _pre_body,
        grid=(2, N_ROW_BLKS),
        in_specs=[
            pl.BlockSpec((ROW_BLK, D), lambda p, i: (i, 0)),
            pl.BlockSpec((1, D, D), lambda p, i: (p, 0, 0)),
            pl.BlockSpec((1, D, 2 * H), lambda p, i: (p, 0, 0)),
            pl.BlockSpec((1, D, 2 * H), lambda p, i: (p, 0, 0)),
        ],
        out_specs=[
            pl.BlockSpec((1, ROW_BLK, D + 2 * H), lambda p, i: (p, i, 0)),
            pl.BlockSpec((1, ROW_BLK, 2 * H), lambda p, i: (p, i, 0)),
        ],
        out_shape=[
            jax.ShapeDtypeStruct((2, N_NODES, D + 2 * H), jnp.float32),
            jax.ShapeDtypeStruct((2, N_NODES, 2 * H), jnp.float32),
        ],
    )(h, Ws, alS, arS)


def _post_body(acc_ref, Rsel_ref, sW1_ref, sb1_ref, w2_ref,
               pW_ref, pb_ref, out_ref):
    Rsel = Rsel_ref[...]
    sW1 = sW1_ref[...]
    sb1 = sb1_ref[...]
    w2 = w2_ref[...]

    def make_z(p):
        blk = acc_ref[p]
        s_rep = jnp.dot(blk[:, D:], Rsel, preferred_element_type=jnp.float32)
        x = blk[:, :D] / (s_rep + 1e-9)
        return jnp.where(x > 0, x, jnp.exp(x) - 1.0)

    z0 = make_z(0)
    z1 = make_z(1)

    def sem_w(z):
        t = jnp.tanh(jnp.dot(z, sW1, preferred_element_type=jnp.float32) + sb1)
        return (t * w2).sum(axis=1, keepdims=True)

    w0 = sem_w(z0)
    w1 = sem_w(z1)
    m = jnp.maximum(w0, w1)
    e0 = jnp.exp(w0 - m)
    e1 = jnp.exp(w1 - m)
    b0 = e0 / (e0 + e1)
    hz = b0 * z0 + (1.0 - b0) * z1
    logits = jnp.dot(hz, pW_ref[...], preferred_element_type=jnp.float32) + pb_ref[...]
    lm = jnp.max(logits, axis=1, keepdims=True)
    lse = jnp.log(jnp.sum(jnp.exp(logits - lm), axis=1, keepdims=True)) + lm
    out_ref[...] = logits - lse


def _tc_post(acc, Rsel, sem_W1, sem_b1, w2row, pred_W, pred_b):
    return pl.pallas_call(
        _post_body,
        grid=(N_ROW_BLKS,),
        in_specs=[
            pl.BlockSpec((2, ROW_BLK, D + 2 * H), lambda i: (0, i, 0)),
            pl.BlockSpec((2 * H, D), lambda i: (0, 0)),
            pl.BlockSpec((D, D), lambda i: (0, 0)),
            pl.BlockSpec((1, D), lambda i: (0, 0)),
            pl.BlockSpec((1, D), lambda i: (0, 0)),
            pl.BlockSpec((D, OUT), lambda i: (0, 0)),
            pl.BlockSpec((1, OUT), lambda i: (0, 0)),
        ],
        out_specs=pl.BlockSpec((ROW_BLK, OUT), lambda i: (i, 0)),
        out_shape=jax.ShapeDtypeStruct((N_NODES, OUT), jnp.float32),
    )(acc, Rsel, sem_W1, sem_b1, w2row, pred_W, pred_b)


CH = 80
CHUNKS_PER_CORE = N_EDGES // CH
TILES = 16
R_MAIN = 624
ZR = 312


def _edge_sc_body(featx2, erx2, idx3, zerX,
                  accx_out,
                  idx_v0, idx_v1, er_v0, er_v1,
                  fx_v0, fx_v1,
                  accS, sem0, sem1, sco0, sco1):
    idx_v = (idx_v0, idx_v1)
    er_v = (er_v0, er_v1)
    fx_v = (fx_v0, fx_v1)
    sems = (sem0, sem1)
    scat_sems = (sco0, sco1)
    c = lax.axis_index("c")
    s = lax.axis_index("s")
    row0 = s * R_MAIN

    for k in range(2):
        pltpu.sync_copy(zerX, accS.at[pl.ds(row0 + k * ZR, ZR), :])

    @pl.when(s == TILES - 1)
    def _zero_tail():
        pltpu.sync_copy(zerX.at[pl.ds(0, 16), :],
                        accS.at[pl.ds(TILES * R_MAIN, 16), :])

    plsc.subcore_barrier()

    nj = CHUNKS_PER_CORE // TILES

    def gather_copies(b):
        return (pltpu.make_async_copy(erx2.at[idx_v[b].at[1]], er_v[b], sems[b]),
                pltpu.make_async_copy(featx2.at[idx_v[b].at[0]], fx_v[b], sems[b]))

    def scatter_start(b):
        pltpu.async_copy(fx_v[b], accS.at[idx_v[b].at[2]], scat_sems[b], add=True)

    def scatter_wait(b):
        pltpu.make_async_copy(fx_v[b], accS.at[idx_v[b].at[2]],
                              scat_sems[b]).wait()

    def fire(step, b):
        @pl.when(step >= 2)
        def _w():
            scatter_wait(b)

        k = c * CHUNKS_PER_CORE + s + step * TILES
        pltpu.sync_copy(idx3.at[k], idx_v[b])
        for cp in gather_copies(b):
            cp.start()

    def consume(step, b):
        for cp in gather_copies(b):
            cp.wait()

        @plsc.parallel_loop(0, CH, unroll=4)
        def edge_body(i):
            x = fx_v[b][i, pl.ds(D, 2 * H)] + er_v[b][i]
            x = jnp.where(x >= 0, x, 0.2 * x)
            ex = jnp.exp(x)
            fx_v[b][i, pl.ds(D, 2 * H)] = ex
            for hh in range(H):
                idx = jnp.full((16, 1), hh, dtype=jnp.int32)
                g = lax.gather(
                    ex, idx,
                    lax.GatherDimensionNumbers(offset_dims=(),
                                               collapsed_slice_dims=(0,),
                                               start_index_map=(0,)),
                    slice_sizes=(1,),
                    mode=lax.GatherScatterMode.PROMISE_IN_BOUNDS)
                fx_v[b][i, pl.ds(hh * DH, DH)] = fx_v[b][i, pl.ds(hh * DH, DH)] * g

        scatter_start(b)

    fire(0, 0)

    def pair_body(jj, carry):
        for b in range(2):
            step = 2 * jj + b

            @pl.when(step + 1 < nj)
            def _f():
                fire(step + 1, 1 - b)

            @pl.when(step < nj)
            def _c():
                consume(step, b)
        return carry

    lax.fori_loop(0, (nj + 1) // 2, pair_body, 0)

    scatter_wait(0)
    scatter_wait(1)

    plsc.subcore_barrier()
    out_row0 = c * N_NODES + row0
    pltpu.sync_copy(accS.at[pl.ds(row0, R_MAIN), :],
                    accx_out.at[pl.ds(out_row0, R_MAIN), :])

    @pl.when(s == TILES - 1)
    def _write_tail():
        t0 = TILES * R_MAIN
        pltpu.sync_copy(accS.at[pl.ds(t0, 16), :],
                        accx_out.at[pl.ds(c * N_NODES + t0, 16), :])


def _edge_pass_sc(featx2, erx2, idx3):
    DX = D + 2 * H
    zerX = jnp.zeros((ZR, DX), jnp.float32)
    mesh = plsc.VectorSubcoreMesh(core_axis_name="c", subcore_axis_name="s")
    f = functools.partial(
        pl.kernel,
        mesh=mesh,
        compiler_params=pltpu.CompilerParams(use_tc_tiling_on_sc=False),
        out_type=[
            jax.ShapeDtypeStruct((2 * N_NODES, DX), jnp.float32),
        ],
        scratch_types=[
            pltpu.VMEM((3, CH), jnp.int32),
            pltpu.VMEM((3, CH), jnp.int32),
            pltpu.VMEM((CH, 2 * H), jnp.float32),
            pltpu.VMEM((CH, 2 * H), jnp.float32),
            pltpu.VMEM((CH, DX), jnp.float32),
            pltpu.VMEM((CH, DX), jnp.float32),
            pltpu.VMEM_SHARED((N_NODES, DX), jnp.float32),
            pltpu.SemaphoreType.DMA,
            pltpu.SemaphoreType.DMA,
            pltpu.SemaphoreType.DMA,
            pltpu.SemaphoreType.DMA,
        ],
    )(_edge_sc_body)
    return f(featx2, erx2, idx3, zerX)[0]


def kernel(h, edge_index_0, edge_index_1, fc_W_0, attn_l_0, attn_r_0,
           fc_W_1, attn_l_1, attn_r_1, sem_W1, sem_b1, sem_W2, sem_b2,
           pred_W, pred_b):
    Ws = jnp.stack([fc_W_0, fc_W_1])
    d_ids = np.arange(D) // DH
    sel = (d_ids[:, None] == (np.arange(2 * H)[None, :] % H)).astype(np.float32)
    sel = jnp.asarray(sel)
    alS = jnp.stack([attn_l_0.reshape(D, 1) * sel, attn_l_1.reshape(D, 1) * sel])
    arS = jnp.stack([attn_r_0.reshape(D, 1) * sel, attn_r_1.reshape(D, 1) * sel])

    srcg = jnp.concatenate([edge_index_0[0], edge_index_1[0] + N_NODES]).astype(jnp.int32)
    dstg = jnp.concatenate([edge_index_0[1], edge_index_1[1] + N_NODES]).astype(jnp.int32)
    dstl = jnp.concatenate([edge_index_0[1], edge_index_1[1]]).astype(jnp.int32)
    idx3 = jnp.stack([srcg.reshape(-1, CH), dstg.reshape(-1, CH),
                      dstl.reshape(-1, CH)], axis=1)

    featx, erx = _tc_pre(h, Ws, alS, arS)

    acc2 = _edge_pass_sc(featx.reshape(2 * N_NODES, D + 2 * H),
                         erx.reshape(2 * N_NODES, 2 * H), idx3)
    acc = acc2.reshape(2, N_NODES, D + 2 * H)

    Rsel = (np.arange(2 * H)[:, None] == (np.arange(D)[None, :] // DH)).astype(np.float32)
    Rsel = jnp.asarray(Rsel)
    return _tc_post(acc, Rsel, sem_W1, sem_b1.reshape(1, D),
                    sem_W2.reshape(1, D), pred_W, pred_b.reshape(1, OUT))


# speedup vs baseline: 1.1707x; 1.1497x over previous
import functools
import jax
import jax.numpy as jnp
import numpy as np
from jax import lax
from jax.experimental import pallas as pl
from jax.experimental.pallas import tpu as pltpu
from jax.experimental.pallas import tpu_sc as plsc

N_NODES = 10000
N_EDGES = 320000
D = 128
H = 8
DH = 16
OUT = 64

ROW_BLK = 1000
N_ROW_BLKS = N_NODES // ROW_BLK


def _pre_body(h_ref, W_ref, alS_ref, arS_ref, featx_ref, erx_ref):
    h = h_ref[...]
    W = W_ref[0]
    feat = jnp.dot(h, W, preferred_element_type=jnp.float32)
    elx = jnp.dot(feat, alS_ref[0], preferred_element_type=jnp.float32)
    featx_ref[0] = jnp.concatenate([feat, elx], axis=1)
    erx_ref[0] = jnp.dot(feat, arS_ref[0], preferred_element_type=jnp.float32)


def _tc_pre(h, Ws, alS, arS):
    return pl.pallas_call(
        _pre_body,
        grid=(2, N_ROW_BLKS),
        in_specs=[
            pl.BlockSpec((ROW_BLK, D), lambda p, i: (i, 0)),
            pl.BlockSpec((1, D, D), lambda p, i: (p, 0, 0)),
            pl.BlockSpec((1, D, 2 * H), lambda p, i: (p, 0, 0)),
            pl.BlockSpec((1, D, 2 * H), lambda p, i: (p, 0, 0)),
        ],
        out_specs=[
            pl.BlockSpec((1, ROW_BLK, D + 2 * H), lambda p, i: (p, i, 0)),
            pl.BlockSpec((1, ROW_BLK, 2 * H), lambda p, i: (p, i, 0)),
        ],
        out_shape=[
            jax.ShapeDtypeStruct((2, N_NODES, D + 2 * H), jnp.float32),
            jax.ShapeDtypeStruct((2, N_NODES, 2 * H), jnp.float32),
        ],
    )(h, Ws, alS, arS)


def _post_body(acc_ref, Rsel_ref, sW1_ref, sb1_ref, w2_ref,
               pW_ref, pb_ref, out_ref):
    Rsel = Rsel_ref[...]
    sW1 = sW1_ref[...]
    sb1 = sb1_ref[...]
    w2 = w2_ref[...]

    def make_z(p):
        blk = acc_ref[p]
        s_rep = jnp.dot(blk[:, D:], Rsel, preferred_element_type=jnp.float32)
        x = blk[:, :D] / (s_rep + 1e-9)
        return jnp.where(x > 0, x, jnp.exp(x) - 1.0)

    z0 = make_z(0)
    z1 = make_z(1)

    def sem_w(z):
        t = jnp.tanh(jnp.dot(z, sW1, preferred_element_type=jnp.float32) + sb1)
        return (t * w2).sum(axis=1, keepdims=True)

    w0 = sem_w(z0)
    w1 = sem_w(z1)
    m = jnp.maximum(w0, w1)
    e0 = jnp.exp(w0 - m)
    e1 = jnp.exp(w1 - m)
    b0 = e0 / (e0 + e1)
    hz = b0 * z0 + (1.0 - b0) * z1
    logits = jnp.dot(hz, pW_ref[...], preferred_element_type=jnp.float32) + pb_ref[...]
    lm = jnp.max(logits, axis=1, keepdims=True)
    lse = jnp.log(jnp.sum(jnp.exp(logits - lm), axis=1, keepdims=True)) + lm
    out_ref[...] = logits - lse


def _tc_post(acc, Rsel, sem_W1, sem_b1, w2row, pred_W, pred_b):
    return pl.pallas_call(
        _post_body,
        grid=(N_ROW_BLKS,),
        in_specs=[
            pl.BlockSpec((2, ROW_BLK, D + 2 * H), lambda i: (0, i, 0)),
            pl.BlockSpec((2 * H, D), lambda i: (0, 0)),
            pl.BlockSpec((D, D), lambda i: (0, 0)),
            pl.BlockSpec((1, D), lambda i: (0, 0)),
            pl.BlockSpec((1, D), lambda i: (0, 0)),
            pl.BlockSpec((D, OUT), lambda i: (0, 0)),
            pl.BlockSpec((1, OUT), lambda i: (0, 0)),
        ],
        out_specs=pl.BlockSpec((ROW_BLK, OUT), lambda i: (i, 0)),
        out_shape=jax.ShapeDtypeStruct((N_NODES, OUT), jnp.float32),
    )(acc, Rsel, sem_W1, sem_b1, w2row, pred_W, pred_b)


CH = 80
CHUNKS_PER_CORE = N_EDGES // CH
TILES = 16
R_MAIN = 624
ZR = 312


def _edge_sc_body(featx2, erx2, idx3, zerX,
                  accx_out,
                  idx_v0, idx_v1, idx_v2, er_v0, er_v1,
                  fx_v0, fx_v1,
                  accS, sem0, sem1, sco0, sco1, si0, si1, si2):
    idx_v = (idx_v0, idx_v1, idx_v2)
    er_v = (er_v0, er_v1)
    fx_v = (fx_v0, fx_v1)
    sems = (sem0, sem1)
    scat_sems = (sco0, sco1)
    idx_sems = (si0, si1, si2)
    c = lax.axis_index("c")
    s = lax.axis_index("s")
    row0 = s * R_MAIN

    for k in range(2):
        pltpu.sync_copy(zerX, accS.at[pl.ds(row0 + k * ZR, ZR), :])

    @pl.when(s == TILES - 1)
    def _zero_tail():
        pltpu.sync_copy(zerX.at[pl.ds(0, 16), :],
                        accS.at[pl.ds(TILES * R_MAIN, 16), :])

    plsc.subcore_barrier()

    nj = CHUNKS_PER_CORE // TILES

    def chunk_of(step):
        return c * CHUNKS_PER_CORE + s + step * TILES

    def idx_copy(step, b3):
        return pltpu.make_async_copy(idx3.at[chunk_of(step)], idx_v[b3],
                                     idx_sems[b3])

    def gather_copies(b2, b3):
        return (pltpu.make_async_copy(erx2.at[idx_v[b3].at[1]], er_v[b2], sems[b2]),
                pltpu.make_async_copy(featx2.at[idx_v[b3].at[0]], fx_v[b2], sems[b2]))

    def scatter_start(b2, b3):
        pltpu.async_copy(fx_v[b2], accS.at[idx_v[b3].at[2]], scat_sems[b2],
                         add=True)

    def scatter_wait(b2, b3):
        pltpu.make_async_copy(fx_v[b2], accS.at[idx_v[b3].at[2]],
                              scat_sems[b2]).wait()

    def fire_idx(step, b3):
        idx_copy(step, b3).start()

    def fire_gather(step, b2, b3):
        @pl.when(step >= 2)
        def _w():
            scatter_wait(b2, (b3 + 1) % 3)

        idx_copy(step, b3).wait()
        for cp in gather_copies(b2, b3):
            cp.start()

    def consume(step, b2, b3):
        for cp in gather_copies(b2, b3):
            cp.wait()

        @plsc.parallel_loop(0, CH, unroll=4)
        def edge_body(i):
            x = fx_v[b2][i, pl.ds(D, 2 * H)] + er_v[b2][i]
            x = jnp.where(x >= 0, x, 0.2 * x)
            ex = jnp.exp(x)
            fx_v[b2][i, pl.ds(D, 2 * H)] = ex
            for hh in range(H):
                idx = jnp.full((16, 1), hh, dtype=jnp.int32)
                g = lax.gather(
                    ex, idx,
                    lax.GatherDimensionNumbers(offset_dims=(),
                                               collapsed_slice_dims=(0,),
                                               start_index_map=(0,)),
                    slice_sizes=(1,),
                    mode=lax.GatherScatterMode.PROMISE_IN_BOUNDS)
                fx_v[b2][i, pl.ds(hh * DH, DH)] = fx_v[b2][i, pl.ds(hh * DH, DH)] * g

        scatter_start(b2, b3)

    fire_idx(0, 0)
    fire_idx(1, 1)
    fire_gather(0, 0, 0)

    def six_body(jj, carry):
        for u in range(6):
            step = 6 * jj + u

            @pl.when(step + 1 < nj)
            def _fg():
                fire_gather(step + 1, (u + 1) % 2, (u + 1) % 3)

            @pl.when(step + 2 < nj)
            def _fi():
                fire_idx(step + 2, (u + 2) % 3)

            @pl.when(step < nj)
            def _c():
                consume(step, u % 2, u % 3)
        return carry

    lax.fori_loop(0, (nj + 5) // 6, six_body, 0)

    scatter_wait((nj - 2) % 2, (nj - 2) % 3)
    scatter_wait((nj - 1) % 2, (nj - 1) % 3)

    plsc.subcore_barrier()
    out_row0 = c * N_NODES + row0
    pltpu.sync_copy(accS.at[pl.ds(row0, R_MAIN), :],
                    accx_out.at[pl.ds(out_row0, R_MAIN), :])

    @pl.when(s == TILES - 1)
    def _write_tail():
        t0 = TILES * R_MAIN
        pltpu.sync_copy(accS.at[pl.ds(t0, 16), :],
                        accx_out.at[pl.ds(c * N_NODES + t0, 16), :])


def _edge_pass_sc(featx2, erx2, idx3):
    DX = D + 2 * H
    zerX = jnp.zeros((ZR, DX), jnp.float32)
    mesh = plsc.VectorSubcoreMesh(core_axis_name="c", subcore_axis_name="s")
    f = functools.partial(
        pl.kernel,
        mesh=mesh,
        compiler_params=pltpu.CompilerParams(use_tc_tiling_on_sc=False),
        out_type=[
            jax.ShapeDtypeStruct((2 * N_NODES, DX), jnp.float32),
        ],
        scratch_types=[
            pltpu.VMEM((3, CH), jnp.int32),
            pltpu.VMEM((3, CH), jnp.int32),
            pltpu.VMEM((3, CH), jnp.int32),
            pltpu.VMEM((CH, 2 * H), jnp.float32),
            pltpu.VMEM((CH, 2 * H), jnp.float32),
            pltpu.VMEM((CH, DX), jnp.float32),
            pltpu.VMEM((CH, DX), jnp.float32),
            pltpu.VMEM_SHARED((N_NODES, DX), jnp.float32),
            pltpu.SemaphoreType.DMA,
            pltpu.SemaphoreType.DMA,
            pltpu.SemaphoreType.DMA,
            pltpu.SemaphoreType.DMA,
            pltpu.SemaphoreType.DMA,
            pltpu.SemaphoreType.DMA,
            pltpu.SemaphoreType.DMA,
        ],
    )(_edge_sc_body)
    return f(featx2, erx2, idx3, zerX)[0]


def kernel(h, edge_index_0, edge_index_1, fc_W_0, attn_l_0, attn_r_0,
           fc_W_1, attn_l_1, attn_r_1, sem_W1, sem_b1, sem_W2, sem_b2,
           pred_W, pred_b):
    Ws = jnp.stack([fc_W_0, fc_W_1])
    d_ids = np.arange(D) // DH
    sel = (d_ids[:, None] == (np.arange(2 * H)[None, :] % H)).astype(np.float32)
    sel = jnp.asarray(sel)
    alS = jnp.stack([attn_l_0.reshape(D, 1) * sel, attn_l_1.reshape(D, 1) * sel])
    arS = jnp.stack([attn_r_0.reshape(D, 1) * sel, attn_r_1.reshape(D, 1) * sel])

    srcg = jnp.concatenate([edge_index_0[0], edge_index_1[0] + N_NODES]).astype(jnp.int32)
    dstg = jnp.concatenate([edge_index_0[1], edge_index_1[1] + N_NODES]).astype(jnp.int32)
    dstl = jnp.concatenate([edge_index_0[1], edge_index_1[1]]).astype(jnp.int32)
    idx3 = jnp.stack([srcg.reshape(-1, CH), dstg.reshape(-1, CH),
                      dstl.reshape(-1, CH)], axis=1)

    featx, erx = _tc_pre(h, Ws, alS, arS)

    acc2 = _edge_pass_sc(featx.reshape(2 * N_NODES, D + 2 * H),
                         erx.reshape(2 * N_NODES, 2 * H), idx3)
    acc = acc2.reshape(2, N_NODES, D + 2 * H)

    Rsel = (np.arange(2 * H)[:, None] == (np.arange(D)[None, :] // DH)).astype(np.float32)
    Rsel = jnp.asarray(Rsel)
    return _tc_post(acc, Rsel, sem_W1, sem_b1.reshape(1, D),
                    sem_W2.reshape(1, D), pred_W, pred_b.reshape(1, OUT))
